# Initial kernel scaffold; baseline (speedup 1.0000x reference)
import jax
import jax.numpy as jnp
from jax.experimental import pallas as pl


def kernel(token_ids, embedding_matrix):
    raise NotImplementedError("write your pallas kernel here")


# speedup vs baseline: 9.7583x; 9.7583x over previous
import functools

import jax
import jax.numpy as jnp
from jax import lax
from jax.experimental import pallas as pl
from jax.experimental.pallas import tpu as pltpu
from jax.experimental.pallas import tpu_sc as plsc

B = 4096
HIST = 50
D = 64
L = 16
NC = 2
NS = 16
NW = NC * NS
PAIRS = B // 2
PPW = PAIRS // NW
IDXPP = 2 * HIST
NBUF = 4
VPR = D // L
INV = 1.0 / HIST

_mesh = plsc.VectorSubcoreMesh(core_axis_name="c", subcore_axis_name="s")


@functools.partial(
    pl.kernel,
    out_type=jax.ShapeDtypeStruct((B * D,), jnp.float32),
    mesh=_mesh,
    compiler_params=pltpu.CompilerParams(use_tc_tiling_on_sc=False),
    scratch_types=[
        pltpu.VMEM((PPW, IDXPP), jnp.int32),
        *[pltpu.VMEM((IDXPP, D), jnp.float32) for _ in range(NBUF)],
        pltpu.VMEM((2 * PPW * D,), jnp.float32),
        *[pltpu.SemaphoreType.DMA for _ in range(NBUF)],
    ],
)
def _emb_mean(tok_hbm, table_hbm, out_hbm, idx_v, rb0, rb1, rb2, rb3,
              out_v, sm0, sm1, sm2, sm3):
    bufs = (rb0, rb1, rb2, rb3)
    sems = (sm0, sm1, sm2, sm3)
    wid = lax.axis_index("s") * NC + lax.axis_index("c")

    pltpu.sync_copy(tok_hbm.at[pl.ds(wid * PPW, PPW), :], idx_v)

    for b in range(NBUF):
        pltpu.make_async_copy(
            table_hbm.at[idx_v.at[b]], bufs[b], sems[b]).start()

    @pl.loop(0, PPW, step=NBUF)
    def _(g0):
        for b in range(NBUF):
            g = g0 + b
            buf, sem = bufs[b], sems[b]
            pltpu.make_async_copy(table_hbm.at[idx_v.at[g]], buf, sem).wait()
            for r in range(2):
                def body(j, acc, _r=r, _buf=buf):
                    row = _buf.at[_r * HIST + j]
                    return tuple(acc[c] + row[pl.ds(c * L, L)]
                                 for c in range(VPR))
                acc = lax.fori_loop(
                    0, HIST, body,
                    tuple(jnp.zeros((L,), jnp.float32) for _ in range(VPR)))
                obase = (2 * g + r) * D
                for c in range(VPR):
                    out_v[pl.ds(obase + c * L, L)] = acc[c] * INV
            nxt = g + NBUF

            @pl.when(nxt < PPW)
            def _():
                pltpu.make_async_copy(
                    table_hbm.at[idx_v.at[nxt]], buf, sem).start()

    pltpu.sync_copy(out_v, out_hbm.at[pl.ds(wid * 2 * PPW * D, 2 * PPW * D)])


def kernel(token_ids, embedding_matrix):
    tok2 = token_ids.reshape(PAIRS, IDXPP)
    out = _emb_mean(tok2, embedding_matrix)
    return out.reshape(B, D)


# speedup vs baseline: 9.9428x; 1.0189x over previous
import functools

import jax
import jax.numpy as jnp
from jax import lax
from jax.experimental import pallas as pl
from jax.experimental.pallas import tpu as pltpu
from jax.experimental.pallas import tpu_sc as plsc

B = 4096
HIST = 50
D = 64
L = 16
NC = 2
NS = 16
NW = NC * NS
PAIRS = B // 2
PPW = PAIRS // NW
IDXPP = 2 * HIST
NBUF = 4
VPR = D // L
INV = 1.0 / HIST

_mesh = plsc.VectorSubcoreMesh(core_axis_name="c", subcore_axis_name="s")


@functools.partial(
    pl.kernel,
    out_type=jax.ShapeDtypeStruct((B * D,), jnp.float32),
    mesh=_mesh,
    compiler_params=pltpu.CompilerParams(use_tc_tiling_on_sc=False),
    scratch_types=[
        pltpu.VMEM((PPW, IDXPP), jnp.int32),
        *[pltpu.VMEM((IDXPP, D), jnp.float32) for _ in range(NBUF)],
        pltpu.VMEM((2 * PPW * D,), jnp.float32),
        *[pltpu.SemaphoreType.DMA for _ in range(NBUF)],
    ],
)
def _emb_mean(tok_hbm, table_hbm, out_hbm, idx_v, rb0, rb1, rb2, rb3,
              out_v, sm0, sm1, sm2, sm3):
    bufs = (rb0, rb1, rb2, rb3)
    sems = (sm0, sm1, sm2, sm3)
    wid = lax.axis_index("s") * NC + lax.axis_index("c")

    pltpu.sync_copy(tok_hbm.at[pl.ds(wid * PPW, PPW), :], idx_v)

    for b in range(NBUF):
        pltpu.make_async_copy(
            table_hbm.at[idx_v.at[b]], bufs[b], sems[b]).start()

    @pl.loop(0, PPW, step=NBUF)
    def _(g0):
        for b in range(NBUF):
            g = g0 + b
            buf, sem = bufs[b], sems[b]
            pltpu.make_async_copy(table_hbm.at[idx_v.at[g]], buf, sem).wait()
            for r in range(2):
                def body(j, acc, _r=r, _buf=buf):
                    row = _buf.at[_r * HIST + j]
                    return tuple(acc[c] + row[pl.ds(c * L, L)]
                                 for c in range(VPR))
                acc = lax.fori_loop(
                    0, HIST, body,
                    tuple(jnp.zeros((L,), jnp.float32) for _ in range(VPR)),
                    unroll=10)
                obase = (2 * g + r) * D
                for c in range(VPR):
                    out_v[pl.ds(obase + c * L, L)] = acc[c] * INV
            nxt = g + NBUF

            @pl.when(nxt < PPW)
            def _():
                pltpu.make_async_copy(
                    table_hbm.at[idx_v.at[nxt]], buf, sem).start()

    pltpu.sync_copy(out_v, out_hbm.at[pl.ds(wid * 2 * PPW * D, 2 * PPW * D)])


def kernel(token_ids, embedding_matrix):
    tok2 = token_ids.reshape(PAIRS, IDXPP)
    out = _emb_mean(tok2, embedding_matrix)
    return out.reshape(B, D)
